# baseline (speedup 1.0000x reference)
import jax
import jax.numpy as jnp
from jax import lax
from jax.experimental import pallas as pl
from jax.experimental.pallas import tpu as pltpu
from jax.experimental.pallas import tpu_sc as plsc

N = 10000
E = 320000
F_IN = 128
H1 = 128
D_EMB = 256
G = 64

NC = 2
NS = 16
CHUNK = 125
EROWS = E // CHUNK
TILE_EROWS = EROWS // (NC * NS)
N_PAD = 10240
NODES_PER_TILE = N_PAD // NS
ZROWS = 128

_f32 = jnp.float32


def _deg_body(dst2d, out, accdeg, dst_loc, ones_v, zbuf):
    cid = lax.axis_index("c")
    sid = lax.axis_index("s")
    wid = cid * NS + sid

    def fill_ones(i, _):
        ones_v[i, :] = jnp.ones((16,), _f32)
        return 0
    lax.fori_loop(0, CHUNK, fill_ones, 0)

    def fill_zero(i, _):
        zbuf[i, :] = jnp.zeros((16,), _f32)
        return 0
    lax.fori_loop(0, ZROWS, fill_zero, 0)

    for t in range(NODES_PER_TILE // ZROWS):
        pltpu.sync_copy(zbuf, accdeg.at[pl.ds(sid * NODES_PER_TILE + t * ZROWS, ZROWS)])
    plsc.subcore_barrier()

    pltpu.sync_copy(dst2d.at[pl.ds(wid * TILE_EROWS, TILE_EROWS)], dst_loc)

    def body(j, _):
        pltpu.sync_copy(ones_v, accdeg.at[dst_loc.at[j]], add=True)
        return 0
    lax.fori_loop(0, TILE_EROWS, body, 0)

    plsc.subcore_barrier()
    for t in range(NODES_PER_TILE // ZROWS):
        sl = pl.ds(sid * NODES_PER_TILE + t * ZROWS, ZROWS)
        pltpu.sync_copy(accdeg.at[sl], out.at[cid, sl])


def _deg_pass(dst2d):
    return pl.kernel(
        _deg_body,
        out_type=jax.ShapeDtypeStruct((NC, N_PAD, 16), _f32),
        mesh=plsc.VectorSubcoreMesh(core_axis_name="c", subcore_axis_name="s"),
        scratch_types=[
            pltpu.VMEM_SHARED((N_PAD, 16), _f32),
            pltpu.VMEM((TILE_EROWS, CHUNK), jnp.int32),
            pltpu.VMEM((CHUNK, 16), _f32),
            pltpu.VMEM((ZROWS, 16), _f32),
        ],
    )(dst2d)


FH = F_IN // 2
TILE_CHUNKS = EROWS // NS
NBUF = 4
QUADS = TILE_CHUNKS // NBUF


def _agg_half(yh, src_all, dst_all, acc, rows, gsems, ssems):
    def quad(q, _):
        for b in range(NBUF):
            c = q * NBUF + b

            @pl.when(q > 0)
            def _():
                pltpu.make_async_copy(
                    rows[b], acc.at[dst_all.at[c - NBUF]], ssems[b]).wait()

            pltpu.async_copy(yh.at[src_all.at[c]], rows[b], gsems[b])
        for b in range(NBUF):
            c = q * NBUF + b
            pltpu.make_async_copy(yh.at[src_all.at[c]], rows[b], gsems[b]).wait()
            pltpu.async_copy(rows[b], acc.at[dst_all.at[c]], ssems[b], add=True)
        return 0
    lax.fori_loop(0, QUADS, quad, 0)
    for b in range(NBUF):
        c = (QUADS - 1) * NBUF + b
        pltpu.make_async_copy(rows[b], acc.at[dst_all.at[c]], ssems[b]).wait()


def _agg_body(y, src2d, dst2d, out, acc, src_all, dst_all,
              r0, r1, r2, r3, g0, g1, g2, g3, s0, s1, s2, s3):
    cid = lax.axis_index("c")
    sid = lax.axis_index("s")
    rows = [r0, r1, r2, r3]
    gsems = [g0, g1, g2, g3]
    ssems = [s0, s1, s2, s3]

    def fill_zero(i, _):
        r0[i // 4, pl.ds((i % 4) * 16, 16)] = jnp.zeros((16,), _f32)
        return 0
    lax.fori_loop(0, CHUNK * 4, fill_zero, 0)

    zb = r0.at[pl.ds(0, 120)]
    for t in range(5):
        pltpu.sync_copy(zb, acc.at[pl.ds(sid * NODES_PER_TILE + t * 120, 120)])
    pltpu.sync_copy(r0.at[pl.ds(0, 40)],
                    acc.at[pl.ds(sid * NODES_PER_TILE + 600, 40)])
    plsc.subcore_barrier()

    base = sid * TILE_CHUNKS
    pltpu.sync_copy(src2d.at[pl.ds(base, TILE_CHUNKS)], src_all)
    pltpu.sync_copy(dst2d.at[pl.ds(base, TILE_CHUNKS)], dst_all)

    @pl.when(cid == 0)
    def _():
        _agg_half(y.at[0], src_all, dst_all, acc, rows, gsems, ssems)

    @pl.when(cid == 1)
    def _():
        _agg_half(y.at[1], src_all, dst_all, acc, rows, gsems, ssems)

    plsc.subcore_barrier()
    for t in range(NODES_PER_TILE // ZROWS):
        sl = pl.ds(sid * NODES_PER_TILE + t * ZROWS, ZROWS)
        pltpu.sync_copy(acc.at[sl], out.at[cid, sl])


def _agg_pass(ysp, src2d, dst2d):
    return pl.kernel(
        _agg_body,
        out_type=jax.ShapeDtypeStruct((NC, N_PAD, FH), _f32),
        mesh=plsc.VectorSubcoreMesh(core_axis_name="c", subcore_axis_name="s"),
        scratch_types=[
            pltpu.VMEM_SHARED((N_PAD, FH), _f32),
            pltpu.VMEM((TILE_CHUNKS, CHUNK), jnp.int32),
            pltpu.VMEM((TILE_CHUNKS, CHUNK), jnp.int32),
            pltpu.VMEM((CHUNK, FH), _f32),
            pltpu.VMEM((CHUNK, FH), _f32),
            pltpu.VMEM((CHUNK, FH), _f32),
            pltpu.VMEM((CHUNK, FH), _f32),
        ] + [pltpu.SemaphoreType.DMA] * 8,
        compiler_params=pltpu.CompilerParams(use_tc_tiling_on_sc=False),
    )(ysp, src2d, dst2d)


_MM_BLK = 1000


def _mm_body(x_ref, w_ref, o_ref):
    o_ref[...] = jnp.dot(x_ref[...], w_ref[...], preferred_element_type=_f32,
                         precision=lax.Precision.HIGHEST)


def _mm(x, w):
    n, k = x.shape
    m = w.shape[1]
    return pl.pallas_call(
        _mm_body,
        grid=(n // _MM_BLK,),
        in_specs=[
            pl.BlockSpec((_MM_BLK, k), lambda i: (i, 0)),
            pl.BlockSpec((k, m), lambda i: (0, 0)),
        ],
        out_specs=pl.BlockSpec((_MM_BLK, m), lambda i: (i, 0)),
        out_shape=jax.ShapeDtypeStruct((n, m), _f32),
    )(x, w)


def _scale_body(deg_ref, xw_ref, y_ref, dinv_ref):
    deg = deg_ref[0] + deg_ref[1] + 1.0
    dinv = lax.rsqrt(deg)
    dinv_ref[...] = dinv
    yfull = xw_ref[...] * dinv[:, :1]
    y_ref[0] = yfull[:, :FH]
    y_ref[1] = yfull[:, FH:]


def _scale(deg16, xw):
    return pl.pallas_call(
        _scale_body,
        grid=(N // _MM_BLK,),
        in_specs=[
            pl.BlockSpec((NC, _MM_BLK, 16), lambda i: (0, i, 0)),
            pl.BlockSpec((_MM_BLK, F_IN), lambda i: (i, 0)),
        ],
        out_specs=[
            pl.BlockSpec((NC, _MM_BLK, FH), lambda i: (0, i, 0)),
            pl.BlockSpec((_MM_BLK, 16), lambda i: (i, 0)),
        ],
        out_shape=[
            jax.ShapeDtypeStruct((NC, N, FH), _f32),
            jax.ShapeDtypeStruct((N, 16), _f32),
        ],
    )(deg16, xw)


def _post1_body(acc_ref, y_ref, dinv_ref, b1_ref, z_ref):
    dinv = dinv_ref[:, :1]
    accfull = jnp.concatenate([acc_ref[0], acc_ref[1]], axis=1)
    yfull = jnp.concatenate([y_ref[0], y_ref[1]], axis=1)
    h = dinv * (accfull + yfull) + b1_ref[...]
    z = dinv * jnp.maximum(h, 0.0)
    z_ref[0] = z[:, :FH]
    z_ref[1] = z[:, FH:]


def _post1(acc1, ysp, dinv16, b1_2d):
    return pl.pallas_call(
        _post1_body,
        grid=(N // _MM_BLK,),
        in_specs=[
            pl.BlockSpec((NC, _MM_BLK, FH), lambda i: (0, i, 0)),
            pl.BlockSpec((NC, _MM_BLK, FH), lambda i: (0, i, 0)),
            pl.BlockSpec((_MM_BLK, 16), lambda i: (i, 0)),
            pl.BlockSpec((1, F_IN), lambda i: (0, 0)),
        ],
        out_specs=pl.BlockSpec((NC, _MM_BLK, FH), lambda i: (0, i, 0)),
        out_shape=jax.ShapeDtypeStruct((NC, N, FH), _f32),
    )(acc1, ysp, dinv16, b1_2d)


def _head_body(acc_ref, z_ref, dinv_ref, batch_ref, w2_ref, b2_ref, wb_ref,
               bb_ref, o_ref, p_acc, c_acc):
    i = pl.program_id(0)

    @pl.when(i == 0)
    def _():
        p_acc[...] = jnp.zeros_like(p_acc)
        c_acc[...] = jnp.zeros_like(c_acc)

    dinv = dinv_ref[:, :1]
    accfull = jnp.concatenate([acc_ref[0], acc_ref[1]], axis=1)
    zfull = jnp.concatenate([z_ref[0], z_ref[1]], axis=1)
    w = dinv * (accfull + zfull)
    b = batch_ref[0, 0]
    gids = lax.broadcasted_iota(jnp.int32, (G, _MM_BLK), 0)
    S = (b[None, :] == gids).astype(_f32)
    p_acc[...] += jnp.dot(S, w, preferred_element_type=_f32,
                          precision=lax.Precision.HIGHEST)
    c_acc[...] += jnp.sum(S, axis=1, keepdims=True)

    @pl.when(i == pl.num_programs(0) - 1)
    def _():
        counts = c_acc[:, :1]
        sums = jnp.dot(p_acc[...], w2_ref[...], preferred_element_type=_f32,
                       precision=lax.Precision.HIGHEST)
        sums = sums + counts * b2_ref[...]
        emb = sums / jnp.maximum(counts, 1.0)
        nrm = jnp.sqrt(jnp.sum(emb * emb, axis=1, keepdims=True))
        emb = emb / jnp.maximum(nrm, 1e-12)
        o_ref[...] = jnp.dot(emb, wb_ref[...], preferred_element_type=_f32,
                             precision=lax.Precision.HIGHEST) + bb_ref[...]


def _head(acc2, z, dinv16, batch2d, W2, b2_2d, Wb, bb_2d):
    return pl.pallas_call(
        _head_body,
        grid=(N // _MM_BLK,),
        in_specs=[
            pl.BlockSpec((NC, _MM_BLK, FH), lambda i: (0, i, 0)),
            pl.BlockSpec((NC, _MM_BLK, FH), lambda i: (0, i, 0)),
            pl.BlockSpec((_MM_BLK, 16), lambda i: (i, 0)),
            pl.BlockSpec((1, 1, _MM_BLK), lambda i: (i, 0, 0)),
            pl.BlockSpec((H1, D_EMB), lambda i: (0, 0)),
            pl.BlockSpec((1, D_EMB), lambda i: (0, 0)),
            pl.BlockSpec((D_EMB, 2), lambda i: (0, 0)),
            pl.BlockSpec((1, 2), lambda i: (0, 0)),
        ],
        out_specs=pl.BlockSpec((G, 2), lambda i: (0, 0)),
        out_shape=jax.ShapeDtypeStruct((G, 2), _f32),
        scratch_shapes=[
            pltpu.VMEM((G, F_IN), _f32),
            pltpu.VMEM((G, 128), _f32),
        ],
    )(acc2, z, dinv16, batch2d, W2, b2_2d, Wb, bb_2d)


def kernel(x, edge_index, batch, W1, b1, W2, b2, Wb, bb):
    src2d = edge_index[0].reshape(EROWS, CHUNK)
    dst2d = edge_index[1].reshape(EROWS, CHUNK)
    batch2d = batch.reshape(N // _MM_BLK, 1, _MM_BLK)
    b1_2d = b1.reshape(1, H1)
    b2_2d = b2.reshape(1, D_EMB)
    bb_2d = bb.reshape(1, 2)

    deg16 = _deg_pass(dst2d)
    xw = _mm(x, W1)
    y, dinv16 = _scale(deg16, xw)
    acc1 = _agg_pass(y, src2d, dst2d)
    z = _post1(acc1, y, dinv16, b1_2d)
    acc2 = _agg_pass(z, src2d, dst2d)
    return _head(acc2, z, dinv16, batch2d, W2, b2_2d, Wb, bb_2d)


# speedup vs baseline: 1.1319x; 1.1319x over previous
import jax
import jax.numpy as jnp
from jax import lax
from jax.experimental import pallas as pl
from jax.experimental.pallas import tpu as pltpu
from jax.experimental.pallas import tpu_sc as plsc

N = 10000
E = 320000
F_IN = 128
H1 = 128
D_EMB = 256
G = 64

NC = 2
NS = 16
CHUNK = 125
EROWS = E // CHUNK
TILE_EROWS = EROWS // (NC * NS)
N_PAD = 10240
NODES_PER_TILE = N_PAD // NS
ZROWS = 128

_f32 = jnp.float32


def _deg_body(dst2d, out, accdeg, dst_loc, ones_v, zbuf, ssem):
    cid = lax.axis_index("c")
    sid = lax.axis_index("s")
    wid = cid * NS + sid

    def fill_ones(i, _):
        ones_v[i, :] = jnp.ones((16,), _f32)
        return 0
    lax.fori_loop(0, CHUNK, fill_ones, 0)

    def fill_zero(i, _):
        zbuf[i, :] = jnp.zeros((16,), _f32)
        return 0
    lax.fori_loop(0, ZROWS, fill_zero, 0)

    for t in range(NODES_PER_TILE // ZROWS):
        pltpu.sync_copy(zbuf, accdeg.at[pl.ds(sid * NODES_PER_TILE + t * ZROWS, ZROWS)])
    plsc.subcore_barrier()

    pltpu.sync_copy(dst2d.at[pl.ds(wid * TILE_EROWS, TILE_EROWS)], dst_loc)

    def body(j, _):
        pltpu.async_copy(ones_v, accdeg.at[dst_loc.at[j]], ssem, add=True)

        @pl.when(j >= 8)
        def _():
            pltpu.make_async_copy(ones_v, accdeg.at[dst_loc.at[j - 8]],
                                  ssem).wait()
        return 0
    lax.fori_loop(0, TILE_EROWS, body, 0)
    for k in range(8):
        pltpu.make_async_copy(ones_v, accdeg.at[dst_loc.at[TILE_EROWS - 8 + k]],
                              ssem).wait()

    plsc.subcore_barrier()
    for t in range(NODES_PER_TILE // ZROWS):
        sl = pl.ds(sid * NODES_PER_TILE + t * ZROWS, ZROWS)
        pltpu.sync_copy(accdeg.at[sl], out.at[cid, sl])


def _deg_pass(dst2d):
    return pl.kernel(
        _deg_body,
        out_type=jax.ShapeDtypeStruct((NC, N_PAD, 16), _f32),
        mesh=plsc.VectorSubcoreMesh(core_axis_name="c", subcore_axis_name="s"),
        scratch_types=[
            pltpu.VMEM_SHARED((N_PAD, 16), _f32),
            pltpu.VMEM((TILE_EROWS, CHUNK), jnp.int32),
            pltpu.VMEM((CHUNK, 16), _f32),
            pltpu.VMEM((ZROWS, 16), _f32),
            pltpu.SemaphoreType.DMA,
        ],
    )(dst2d)


GRP = 40
NGRP = TILE_EROWS // GRP
PAIRS = GRP // 2


def _agg_body(y, src2d, dst2d, out, acc, src_g, dst_g, rows0, rows1,
              sem0, sem1):
    cid = lax.axis_index("c")
    sid = lax.axis_index("s")
    wid = cid * NS + sid

    def fill_zero(i, _):
        rows0[i // 8, pl.ds((i % 8) * 16, 16)] = jnp.zeros((16,), _f32)
        return 0
    lax.fori_loop(0, ZROWS * 8, fill_zero, 0)

    for t in range(NODES_PER_TILE // ZROWS):
        pltpu.sync_copy(rows0, acc.at[pl.ds(sid * NODES_PER_TILE + t * ZROWS, ZROWS)])
    plsc.subcore_barrier()

    r0 = rows0.at[pl.ds(0, CHUNK)]
    base = wid * TILE_EROWS
    for g in range(NGRP):
        gb = base + g * GRP
        pltpu.sync_copy(src2d.at[pl.ds(gb, GRP)], src_g)
        pltpu.sync_copy(dst2d.at[pl.ds(gb, GRP)], dst_g)
        pltpu.async_copy(y.at[src_g.at[0]], r0, sem0)

        def body(p, _):
            c0 = 2 * p
            c1 = c0 + 1
            d1 = pltpu.async_copy(y.at[src_g.at[c1]], rows1, sem1)
            pltpu.make_async_copy(y.at[src_g.at[c0]], r0, sem0).wait()
            pltpu.sync_copy(r0, acc.at[dst_g.at[c0]], add=True)

            @pl.when(c0 + 2 < GRP)
            def _():
                pltpu.async_copy(y.at[src_g.at[c0 + 2]], r0, sem0)

            d1.wait()
            pltpu.sync_copy(rows1, acc.at[dst_g.at[c1]], add=True)
            return 0
        lax.fori_loop(0, PAIRS, body, 0)

    plsc.subcore_barrier()
    for t in range(NODES_PER_TILE // ZROWS):
        sl = pl.ds(sid * NODES_PER_TILE + t * ZROWS, ZROWS)
        pltpu.sync_copy(acc.at[sl], out.at[cid, sl])


def _agg_pass(y, src2d, dst2d):
    return pl.kernel(
        _agg_body,
        out_type=jax.ShapeDtypeStruct((NC, N_PAD, F_IN), _f32),
        mesh=plsc.VectorSubcoreMesh(core_axis_name="c", subcore_axis_name="s"),
        scratch_types=[
            pltpu.VMEM_SHARED((N_PAD, F_IN), _f32),
            pltpu.VMEM((GRP, CHUNK), jnp.int32),
            pltpu.VMEM((GRP, CHUNK), jnp.int32),
            pltpu.VMEM((ZROWS, F_IN), _f32),
            pltpu.VMEM((CHUNK, F_IN), _f32),
            pltpu.SemaphoreType.DMA,
            pltpu.SemaphoreType.DMA,
        ],
    )(y, src2d, dst2d)


_MM_BLK = 1000


def _mm_body(x_ref, w_ref, o_ref):
    o_ref[...] = jnp.dot(x_ref[...], w_ref[...], preferred_element_type=_f32,
                         precision=lax.Precision.HIGHEST)


def _mm(x, w):
    n, k = x.shape
    m = w.shape[1]
    return pl.pallas_call(
        _mm_body,
        grid=(n // _MM_BLK,),
        in_specs=[
            pl.BlockSpec((_MM_BLK, k), lambda i: (i, 0)),
            pl.BlockSpec((k, m), lambda i: (0, 0)),
        ],
        out_specs=pl.BlockSpec((_MM_BLK, m), lambda i: (i, 0)),
        out_shape=jax.ShapeDtypeStruct((n, m), _f32),
    )(x, w)


def _mmscale_body(deg_ref, x_ref, w_ref, y_ref, dinv_ref):
    deg = deg_ref[0] + deg_ref[1] + 1.0
    dinv = lax.rsqrt(deg)
    dinv_ref[...] = dinv
    xw = jnp.dot(x_ref[...], w_ref[...], preferred_element_type=_f32,
                 precision=lax.Precision.HIGHEST)
    y_ref[...] = xw * dinv[:, :1]


def _mmscale(deg16, x, W1):
    return pl.pallas_call(
        _mmscale_body,
        grid=(N // _MM_BLK,),
        in_specs=[
            pl.BlockSpec((NC, _MM_BLK, 16), lambda i: (0, i, 0)),
            pl.BlockSpec((_MM_BLK, F_IN), lambda i: (i, 0)),
            pl.BlockSpec((F_IN, H1), lambda i: (0, 0)),
        ],
        out_specs=[
            pl.BlockSpec((_MM_BLK, F_IN), lambda i: (i, 0)),
            pl.BlockSpec((_MM_BLK, 16), lambda i: (i, 0)),
        ],
        out_shape=[
            jax.ShapeDtypeStruct((N, F_IN), _f32),
            jax.ShapeDtypeStruct((N, 16), _f32),
        ],
    )(deg16, x, W1)


def _post1_body(acc_ref, y_ref, dinv_ref, b1_ref, z_ref):
    dinv = dinv_ref[:, :1]
    h = dinv * (acc_ref[0] + acc_ref[1] + y_ref[...]) + b1_ref[...]
    z_ref[...] = dinv * jnp.maximum(h, 0.0)


def _post1(acc1, y, dinv16, b1_2d):
    return pl.pallas_call(
        _post1_body,
        grid=(N // _MM_BLK,),
        in_specs=[
            pl.BlockSpec((NC, _MM_BLK, F_IN), lambda i: (0, i, 0)),
            pl.BlockSpec((_MM_BLK, F_IN), lambda i: (i, 0)),
            pl.BlockSpec((_MM_BLK, 16), lambda i: (i, 0)),
            pl.BlockSpec((1, F_IN), lambda i: (0, 0)),
        ],
        out_specs=pl.BlockSpec((_MM_BLK, F_IN), lambda i: (i, 0)),
        out_shape=jax.ShapeDtypeStruct((N, F_IN), _f32),
    )(acc1, y, dinv16, b1_2d)


def _head_body(acc_ref, z_ref, dinv_ref, batch_ref, w2_ref, b2_ref, wb_ref,
               bb_ref, o_ref, p_acc, c_acc):
    i = pl.program_id(0)

    @pl.when(i == 0)
    def _():
        p_acc[...] = jnp.zeros_like(p_acc)
        c_acc[...] = jnp.zeros_like(c_acc)

    dinv = dinv_ref[:, :1]
    w = dinv * (acc_ref[0] + acc_ref[1] + z_ref[...])
    b = batch_ref[0, 0]
    gids = lax.broadcasted_iota(jnp.int32, (G, _MM_BLK), 0)
    S = (b[None, :] == gids).astype(_f32)
    p_acc[...] += jnp.dot(S, w, preferred_element_type=_f32,
                          precision=lax.Precision.HIGHEST)
    c_acc[...] += jnp.sum(S, axis=1, keepdims=True)

    @pl.when(i == pl.num_programs(0) - 1)
    def _():
        counts = c_acc[:, :1]
        sums = jnp.dot(p_acc[...], w2_ref[...], preferred_element_type=_f32,
                       precision=lax.Precision.HIGHEST)
        sums = sums + counts * b2_ref[...]
        emb = sums / jnp.maximum(counts, 1.0)
        nrm = jnp.sqrt(jnp.sum(emb * emb, axis=1, keepdims=True))
        emb = emb / jnp.maximum(nrm, 1e-12)
        o_ref[...] = jnp.dot(emb, wb_ref[...], preferred_element_type=_f32,
                             precision=lax.Precision.HIGHEST) + bb_ref[...]


def _head(acc2, z, dinv16, batch2d, W2, b2_2d, Wb, bb_2d):
    return pl.pallas_call(
        _head_body,
        grid=(N // _MM_BLK,),
        in_specs=[
            pl.BlockSpec((NC, _MM_BLK, F_IN), lambda i: (0, i, 0)),
            pl.BlockSpec((_MM_BLK, F_IN), lambda i: (i, 0)),
            pl.BlockSpec((_MM_BLK, 16), lambda i: (i, 0)),
            pl.BlockSpec((1, 1, _MM_BLK), lambda i: (i, 0, 0)),
            pl.BlockSpec((H1, D_EMB), lambda i: (0, 0)),
            pl.BlockSpec((1, D_EMB), lambda i: (0, 0)),
            pl.BlockSpec((D_EMB, 2), lambda i: (0, 0)),
            pl.BlockSpec((1, 2), lambda i: (0, 0)),
        ],
        out_specs=pl.BlockSpec((G, 2), lambda i: (0, 0)),
        out_shape=jax.ShapeDtypeStruct((G, 2), _f32),
        scratch_shapes=[
            pltpu.VMEM((G, F_IN), _f32),
            pltpu.VMEM((G, 128), _f32),
        ],
    )(acc2, z, dinv16, batch2d, W2, b2_2d, Wb, bb_2d)


def kernel(x, edge_index, batch, W1, b1, W2, b2, Wb, bb):
    src2d = edge_index[0].reshape(EROWS, CHUNK)
    dst2d = edge_index[1].reshape(EROWS, CHUNK)
    batch2d = batch.reshape(N // _MM_BLK, 1, _MM_BLK)
    b1_2d = b1.reshape(1, H1)
    b2_2d = b2.reshape(1, D_EMB)
    bb_2d = bb.reshape(1, 2)

    deg16 = _deg_pass(dst2d)
    y, dinv16 = _mmscale(deg16, x, W1)
    acc1 = _agg_pass(y, src2d, dst2d)
    z = _post1(acc1, y, dinv16, b1_2d)
    acc2 = _agg_pass(z, src2d, dst2d)
    return _head(acc2, z, dinv16, batch2d, W2, b2_2d, Wb, bb_2d)
